# baseline (speedup 1.0000x reference)
import functools

import jax
import jax.numpy as jnp
from jax import lax
from jax.experimental import pallas as pl
from jax.experimental.pallas import tpu as pltpu
from jax.experimental.pallas import tpu_sc as plsc

_VOCAB = 1000000
_D = 64
_B = 4096
_H = 50
_N = _B * _H
_SCALE = 8.0

_NC = 2
_NS = 16
_NW = _NC * _NS
_ROWS_PER_W = _N // _NW
_G = 128
_GPC = 5
_C = _G * _GPC
_CHUNKS = _ROWS_PER_W // _C


def _scale_chunk(rows):
    @plsc.parallel_loop(0, _C, 1, unroll=8)
    def _(i):
        for j in range(_D // 16):
            sl = pl.ds(j * 16, 16)
            rows[i, sl] = rows[i, sl] * _SCALE


@functools.partial(jax.jit, donate_argnums=())
def _sc_embed(idx2, table):
    mesh = plsc.VectorSubcoreMesh(
        core_axis_name="c", subcore_axis_name="s",
        num_cores=_NC, num_subcores=_NS)

    @functools.partial(
        pl.kernel,
        out_type=jax.ShapeDtypeStruct((_N, _D), jnp.float32),
        mesh=mesh,
        scratch_types=[
            pltpu.VMEM((_CHUNKS * _GPC, _G), jnp.int32),
            pltpu.VMEM((2, _C, _D), jnp.float32),
            pltpu.SemaphoreType.DMA,
            pltpu.SemaphoreType.DMA,
            pltpu.SemaphoreType.DMA,
            pltpu.SemaphoreType.DMA,
        ],
        compiler_params=pltpu.CompilerParams(use_tc_tiling_on_sc=False),
    )
    def k(idx_hbm, tab_hbm, out_hbm, idx_v, rows_v, gsem0, gsem1, osem0, osem1):
        gsems = (gsem0, gsem1)
        osems = (osem0, osem1)
        wid = lax.axis_index("s") * _NC + lax.axis_index("c")
        rbase = wid * _ROWS_PER_W
        pltpu.sync_copy(idx_hbm.at[wid], idx_v)

        def start_gather(g, buf):
            descs = []
            for j in range(_GPC):
                descs.append(pltpu.async_copy(
                    tab_hbm.at[idx_v.at[g * _GPC + j]],
                    rows_v.at[buf, pl.ds(j * _G, _G)],
                    gsems[buf]))
            return descs

        g_descs = {0: start_gather(0, 0)}
        w_descs = {}
        for g in range(_CHUNKS):
            buf = g & 1
            if g + 1 < _CHUNKS:
                if g - 1 in w_descs:
                    w_descs[g - 1].wait()
                g_descs[g + 1] = start_gather(g + 1, buf ^ 1)
            for d in g_descs[g]:
                d.wait()
            _scale_chunk(rows_v.at[buf])
            w_descs[g] = pltpu.async_copy(
                rows_v.at[buf], out_hbm.at[pl.ds(rbase + g * _C, _C)],
                osems[buf])
        for g in (_CHUNKS - 2, _CHUNKS - 1):
            if g >= 0:
                w_descs[g].wait()

    return k(idx2, table)


def kernel(inputs, embeddings):
    idx = inputs.astype(jnp.int32).reshape(_NW, _CHUNKS * _GPC, _G)
    out = _sc_embed(idx, embeddings)
    return out.reshape(_B, _H, _D)


# speedup vs baseline: 1.0019x; 1.0019x over previous
import functools

import jax
import jax.numpy as jnp
from jax import lax
from jax.experimental import pallas as pl
from jax.experimental.pallas import tpu as pltpu
from jax.experimental.pallas import tpu_sc as plsc

_VOCAB = 1000000
_D = 64
_B = 4096
_H = 50
_SCALE = 8.0

_NC = 2
_NS = 16
_NW = _NC * _NS
_IR_PER_W = _B // _NW
_RPC = 16
_CHUNKS = _IR_PER_W // _RPC


def _scale_chunk(rows):
    @plsc.parallel_loop(0, _RPC * _H, 1, unroll=4)
    def _(i):
        r = i // _H
        c = i - r * _H
        for j in range(_D // 16):
            sl = pl.ds(j * 16, 16)
            rows[r, c, sl] = rows[r, c, sl] * _SCALE


@jax.jit
def _sc_embed(idx, table):
    mesh = plsc.VectorSubcoreMesh(
        core_axis_name="c", subcore_axis_name="s",
        num_cores=_NC, num_subcores=_NS)

    @functools.partial(
        pl.kernel,
        out_type=jax.ShapeDtypeStruct((_B, _H, _D), jnp.float32),
        mesh=mesh,
        scratch_types=[
            pltpu.VMEM((_IR_PER_W, _H), jnp.int32),
            pltpu.VMEM((2, _RPC, _H, _D), jnp.float32),
            pltpu.SemaphoreType.DMA,
            pltpu.SemaphoreType.DMA,
            pltpu.SemaphoreType.DMA,
            pltpu.SemaphoreType.DMA,
        ],
        compiler_params=pltpu.CompilerParams(use_tc_tiling_on_sc=False),
    )
    def k(idx_hbm, tab_hbm, out_hbm, idx_v, rows_v, gsem0, gsem1, osem0, osem1):
        gsems = (gsem0, gsem1)
        osems = (osem0, osem1)
        wid = lax.axis_index("s") * _NC + lax.axis_index("c")
        irbase = wid * _IR_PER_W
        pltpu.sync_copy(idx_hbm.at[pl.ds(irbase, _IR_PER_W)], idx_v)

        def start_gather(g, buf):
            descs = []
            for j in range(_RPC):
                descs.append(pltpu.async_copy(
                    tab_hbm.at[idx_v.at[g * _RPC + j]],
                    rows_v.at[buf, j],
                    gsems[buf]))
            return descs

        g_descs = {0: start_gather(0, 0)}
        w_descs = {}
        for g in range(_CHUNKS):
            buf = g & 1
            if g + 1 < _CHUNKS:
                if g - 1 in w_descs:
                    w_descs[g - 1].wait()
                g_descs[g + 1] = start_gather(g + 1, buf ^ 1)
            for d in g_descs[g]:
                d.wait()
            _scale_chunk(rows_v.at[buf])
            w_descs[g] = pltpu.async_copy(
                rows_v.at[buf],
                out_hbm.at[pl.ds(irbase + g * _RPC, _RPC)],
                osems[buf])
        for g in (_CHUNKS - 2, _CHUNKS - 1):
            if g >= 0:
                w_descs[g].wait()

    return k(idx, table)


def kernel(inputs, embeddings):
    return _sc_embed(inputs.astype(jnp.int32), embeddings)


# speedup vs baseline: 1.0606x; 1.0586x over previous
import functools

import jax
import jax.numpy as jnp
from jax import lax
from jax.experimental import pallas as pl
from jax.experimental.pallas import tpu as pltpu
from jax.experimental.pallas import tpu_sc as plsc

_VOCAB = 1000000
_D = 64
_B = 4096
_H = 50
_N = _B * _H
_SCALE = 8.0

_NC = 2
_NS = 16
_NW = _NC * _NS
_C = 128
_CHUNKS = _N // (_NW * _C)


@jax.jit
def _sc_embed(idx3, tabp):
    mesh = plsc.VectorSubcoreMesh(
        core_axis_name="c", subcore_axis_name="s",
        num_cores=_NC, num_subcores=_NS)

    @functools.partial(
        pl.kernel,
        out_type=jax.ShapeDtypeStruct((_N, _D), jnp.float32),
        mesh=mesh,
        scratch_types=[
            pltpu.VMEM((_CHUNKS, _C), jnp.int32),
            pltpu.VMEM((2, _C, 2 * _D), jnp.float32),
            pltpu.VMEM((2, _C, _D), jnp.float32),
            pltpu.SemaphoreType.DMA,
            pltpu.SemaphoreType.DMA,
            pltpu.SemaphoreType.DMA,
            pltpu.SemaphoreType.DMA,
        ],
        compiler_params=pltpu.CompilerParams(use_tc_tiling_on_sc=False),
    )
    def k(idx_hbm, tab_hbm, out_hbm, idx_v, rows_v, out_v,
          gsem0, gsem1, osem0, osem1):
        gsems = (gsem0, gsem1)
        osems = (osem0, osem1)
        wid = lax.axis_index("s") * _NC + lax.axis_index("c")
        pltpu.sync_copy(idx_hbm.at[wid], idx_v)

        rbase0 = pl.multiple_of(wid * (_CHUNKS * _C), _C)

        def start_gather(ck, b):
            pltpu.async_copy(tab_hbm.at[idx_v.at[ck]], rows_v.at[b], gsems[b])

        def wait_gather(ck, b):
            pltpu.make_async_copy(
                tab_hbm.at[idx_v.at[ck]], rows_v.at[b], gsems[b]).wait()

        def wb_slice(ck):
            rb = pl.multiple_of(rbase0 + ck * _C, _C)
            return out_hbm.at[pl.ds(rb, _C)]

        def extract(b):
            rows = rows_v.at[b]
            outb = out_v.at[b]

            @plsc.parallel_loop(0, _C, 1, unroll=4)
            def _(j):
                for g in range(_D // 16):
                    sl = pl.ds(16 * g, 16)
                    outb[j, sl] = rows[j, sl] * _SCALE

        start_gather(0, 0)
        start_gather(1, 1)

        @pl.loop(0, _CHUNKS // 2)
        def _(g2):
            for b in (0, 1):
                ck = 2 * g2 + b
                wait_gather(ck, b)

                @pl.when(ck >= 2)
                def _():
                    pltpu.make_async_copy(
                        out_v.at[b], wb_slice(ck - 2), osems[b]).wait()

                extract(b)
                pltpu.async_copy(out_v.at[b], wb_slice(ck), osems[b])

                @pl.when(ck + 2 < _CHUNKS)
                def _():
                    start_gather(ck + 2, b)

        pltpu.make_async_copy(
            out_v.at[0], wb_slice(_CHUNKS - 2), osems[0]).wait()
        pltpu.make_async_copy(
            out_v.at[1], wb_slice(_CHUNKS - 1), osems[1]).wait()

    return k(idx3, tabp)


def kernel(inputs, embeddings):
    idx3 = inputs.astype(jnp.int32).reshape(_N).reshape(_NW, _CHUNKS, _C)
    tabp = jnp.pad(embeddings, ((0, 0), (0, _D)))
    out = _sc_embed(idx3, tabp)
    return out.reshape(_B, _H, _D)
